# Initial kernel scaffold; baseline (speedup 1.0000x reference)
import jax
import jax.numpy as jnp
from jax.experimental import pallas as pl


def kernel(input_ids, embed_table):
    raise NotImplementedError("write your pallas kernel here")


# speedup vs baseline: 4.8383x; 4.8383x over previous
import functools

import jax
import jax.numpy as jnp
from jax import lax
from jax.experimental import pallas as pl
from jax.experimental.pallas import tpu as pltpu
from jax.experimental.pallas import tpu_sc as plsc


def _embed_lookup(idx_flat, table, B, V, D):
    info = plsc.get_sparse_core_info()
    NC, NS = info.num_cores, info.num_subcores
    NW = NC * NS
    CB = 128
    b_per_w = B // NW
    n_chunks = b_per_w // CB
    mesh = plsc.VectorSubcoreMesh(core_axis_name="c", subcore_axis_name="s")

    @functools.partial(
        pl.kernel,
        mesh=mesh,
        out_type=jax.ShapeDtypeStruct((B, D), jnp.float32),
        scratch_types=[
            pltpu.VMEM((CB,), jnp.int32),
            pltpu.VMEM((CB, D), jnp.float32),
            pltpu.SemaphoreType.DMA,
        ],
    )
    def emb(idx_hbm, table_hbm, out_hbm, idx_v, rows_v, sem):
        wid = lax.axis_index("s") * NC + lax.axis_index("c")
        base = wid * b_per_w

        def body(g, carry):
            off = base + g * CB
            pltpu.sync_copy(idx_hbm.at[pl.ds(off, CB)], idx_v)
            pltpu.async_copy(table_hbm.at[idx_v], rows_v, sem).wait()
            pltpu.sync_copy(rows_v, out_hbm.at[pl.ds(off, CB)])
            return carry

        lax.fori_loop(0, n_chunks, body, 0)

    return emb(idx_flat, table)


def kernel(input_ids, embed_table):
    V, D = embed_table.shape
    B = input_ids.size
    idx_flat = input_ids.reshape((B,)).astype(jnp.int32)
    out = _embed_lookup(idx_flat, embed_table, B, V, D)
    return out.reshape(input_ids.shape + (D,))


# speedup vs baseline: 6.5305x; 1.3498x over previous
import functools

import jax
import jax.numpy as jnp
from jax import lax
from jax.experimental import pallas as pl
from jax.experimental.pallas import tpu as pltpu
from jax.experimental.pallas import tpu_sc as plsc

_CB = 128
_NBUF = 4


def _embed_lookup(idx2d, table, B, V, D):
    info = plsc.get_sparse_core_info()
    NC, NS = info.num_cores, info.num_subcores
    NW = NC * NS
    b_per_w = B // NW
    n_chunks = b_per_w // _CB
    n_outer = n_chunks // _NBUF
    mesh = plsc.VectorSubcoreMesh(core_axis_name="c", subcore_axis_name="s")

    @functools.partial(
        pl.kernel,
        mesh=mesh,
        out_type=jax.ShapeDtypeStruct((B, D), jnp.float32),
        scratch_types=[
            pltpu.VMEM((n_chunks, _CB), jnp.int32),
            pltpu.VMEM((_NBUF, _CB, D), jnp.float32),
            pltpu.SemaphoreType.DMA,
            pltpu.SemaphoreType.DMA,
            pltpu.SemaphoreType.DMA,
            pltpu.SemaphoreType.DMA,
        ],
    )
    def emb(idx_hbm, table_hbm, out_hbm, idx_all, rows, s0, s1, s2, s3):
        sems = [s0, s1, s2, s3]
        wid = lax.axis_index("s") * NC + lax.axis_index("c")
        base = wid * b_per_w
        pltpu.sync_copy(idx_hbm.at[pl.ds(wid * n_chunks, n_chunks)], idx_all)
        for b in range(_NBUF):
            pltpu.async_copy(table_hbm.at[idx_all.at[b]], rows.at[b], sems[b])

        def outer(k, carry):
            g0 = k * _NBUF
            for b in range(_NBUF):
                g = g0 + b
                pltpu.make_async_copy(
                    table_hbm.at[idx_all.at[b]], rows.at[b], sems[b]
                ).wait()
                pltpu.sync_copy(rows.at[b], out_hbm.at[pl.ds(base + g * _CB, _CB)])

                @pl.when(k < n_outer - 1)
                def _():
                    pltpu.async_copy(
                        table_hbm.at[idx_all.at[g + _NBUF]], rows.at[b], sems[b]
                    )

            return carry

        lax.fori_loop(0, n_outer, outer, 0)

    return emb(idx2d, table)


def kernel(input_ids, embed_table):
    V, D = embed_table.shape
    B = input_ids.size
    idx2d = input_ids.reshape((B // _CB, _CB)).astype(jnp.int32)
    out = _embed_lookup(idx2d, embed_table, B, V, D)
    return out.reshape(input_ids.shape + (D,))


# speedup vs baseline: 16.0665x; 2.4602x over previous
import functools

import jax
import jax.numpy as jnp
from jax import lax
from jax.experimental import pallas as pl
from jax.experimental.pallas import tpu as pltpu
from jax.experimental.pallas import tpu_sc as plsc

_CB = 128
_NBUF = 4


def _embed_lookup(idx2d, table, B, V, D):
    info = plsc.get_sparse_core_info()
    NC, NS = info.num_cores, info.num_subcores
    NW = NC * NS
    b_per_w = B // NW
    n_chunks = b_per_w // _CB
    n_outer = n_chunks // _NBUF
    mesh = plsc.VectorSubcoreMesh(core_axis_name="c", subcore_axis_name="s")

    @functools.partial(
        pl.kernel,
        mesh=mesh,
        out_type=jax.ShapeDtypeStruct((B, D), jnp.float32),
        scratch_types=[
            pltpu.VMEM((n_chunks, _CB), jnp.int32),
            pltpu.VMEM((_NBUF, _CB, D), jnp.float32),
            pltpu.VMEM_SHARED((V, D), jnp.float32),
            pltpu.SemaphoreType.DMA,
            pltpu.SemaphoreType.DMA,
            pltpu.SemaphoreType.DMA,
            pltpu.SemaphoreType.DMA,
        ],
    )
    def emb(idx_hbm, table_hbm, out_hbm, idx_all, rows, table_sh, s0, s1, s2, s3):
        sems = [s0, s1, s2, s3]
        sid = lax.axis_index("s")
        wid = sid * NC + lax.axis_index("c")
        base = wid * b_per_w

        @pl.when(sid == 0)
        def _():
            pltpu.sync_copy(table_hbm, table_sh)

        pltpu.sync_copy(idx_hbm.at[pl.ds(wid * n_chunks, n_chunks)], idx_all)
        plsc.subcore_barrier()
        for b in range(_NBUF):
            pltpu.async_copy(table_sh.at[idx_all.at[b]], rows.at[b], sems[b])

        def outer(k, carry):
            g0 = k * _NBUF
            for b in range(_NBUF):
                g = g0 + b
                pltpu.make_async_copy(
                    table_sh.at[idx_all.at[b]], rows.at[b], sems[b]
                ).wait()
                pltpu.sync_copy(rows.at[b], out_hbm.at[pl.ds(base + g * _CB, _CB)])

                @pl.when(k < n_outer - 1)
                def _():
                    pltpu.async_copy(
                        table_sh.at[idx_all.at[g + _NBUF]], rows.at[b], sems[b]
                    )

            return carry

        lax.fori_loop(0, n_outer, outer, 0)

    return emb(idx2d, table)


def kernel(input_ids, embed_table):
    V, D = embed_table.shape
    B = input_ids.size
    idx2d = input_ids.reshape((B // _CB, _CB)).astype(jnp.int32)
    out = _embed_lookup(idx2d, embed_table, B, V, D)
    return out.reshape(input_ids.shape + (D,))
